# baseline (speedup 1.0000x reference)
import functools

import jax
import jax.numpy as jnp
from jax import lax
from jax.experimental import pallas as pl
from jax.experimental.pallas import tpu as pltpu
from jax.experimental.pallas import tpu_sc as plsc

ROWS = 128
COLS = 32768
K = 10
BIG = 0x7FFFFFFF
S_BASE = 1 << 30

_info = plsc.get_sparse_core_info()
NC = _info.num_cores
NS = _info.num_subcores
L = _info.num_lanes
NW = NC * NS
BAND = 8
NBAND = ROWS // BAND
COLS_W = COLS // (NW // NBAND)
NCHUNK = 4
CCOLS = COLS_W // NCHUNK
VECS = CCOLS // L
UNROLL = 16
NSTREAM = 8
VPRS = VECS // NSTREAM
RESCAN_UNROLL = 8
CWORDS = NCHUNK * L


def _sc_topk_body(inp_ref, pk_ref, buf0, buf1, stage, sem0, sem1):
    iota = lax.iota(jnp.int32, L)
    wid = lax.axis_index("s") * NC + lax.axis_index("c")
    band = wid // 2
    row0 = band * BAND
    col0 = (wid & 1) * COLS_W
    bufs = (buf0, buf1)
    sems = (sem0, sem1)

    def chunk_src(c):
        return inp_ref.at[pl.ds(row0, BAND), pl.ds(col0 + c * CCOLS, CCOLS)]

    copies = [None] * NCHUNK
    copies[0] = pltpu.make_async_copy(chunk_src(0), bufs[0], sems[0])
    copies[0].start()

    for c in range(NCHUNK):
        if c + 1 < NCHUNK:
            copies[c + 1] = pltpu.make_async_copy(
                chunk_src(c + 1), bufs[(c + 1) & 1], sems[(c + 1) & 1])
            copies[c + 1].start()
        copies[c].wait()
        buf = bufs[c & 1]
        ccol0 = col0 + c * CCOLS

        GPR = VECS // UNROLL

        def scan_step(i, carry, buf=buf, ccol0=ccol0):
            ms, ixs, m2s = carry
            ms, ixs, m2s = list(ms), list(ixs), list(m2s)
            s = i // GPR
            colb = (i % GPR) * (L * UNROLL)
            gvec = (row0 + s) * COLS + ccol0 + colb + iota
            for u in range(UNROLL):
                p = u % NSTREAM
                v = buf[s, pl.ds(colb + u * L, L)]
                upd = v > ms[p]
                m2s[p] = jnp.maximum(m2s[p], jnp.minimum(v, ms[p]))
                ms[p] = jnp.where(upd, v, ms[p])
                ixs[p] = jnp.where(upd, gvec, ixs[p])
                gvec = gvec + L
            return tuple(ms), tuple(ixs), tuple(m2s)

        ms = tuple(jnp.full((L,), -1.0, jnp.float32)
                   for _ in range(NSTREAM))
        ixs = tuple(jnp.full((L,), BIG, jnp.int32) for _ in range(NSTREAM))
        m2s = tuple(jnp.full((L,), -1.0, jnp.float32)
                    for _ in range(NSTREAM))
        ms, ixs, m2s = lax.fori_loop(0, BAND * GPR, scan_step,
                                     (ms, ixs, m2s))

        def pair_merge(m1, i1, m2, i2):
            tk = (m2 > m1) | ((m2 == m1) & (i2 < i1))
            return jnp.where(tk, m2, m1), jnp.where(tk, i2, i1)

        def merge_all(ms, ixs):
            mC, iC = ms[0], ixs[0]
            for p in range(1, NSTREAM):
                mC, iC = pair_merge(mC, iC, ms[p], ixs[p])
            return mC, iC

        def refresh(pstar, ms, ixs, m2s, buf=buf, ccol0=ccol0):
            svid = pstar * L

            def rstep(k, carry):
                am, ai, a2 = carry
                nm, ni, n2 = [], [], []
                s2 = k // (VPRS // RESCAN_UNROLL)
                jb = (k % (VPRS // RESCAN_UNROLL)) * RESCAN_UNROLL
                gb = (row0 + s2) * COLS + ccol0 + svid + iota
                for u in range(RESCAN_UNROLL):
                    col = svid + (jb + u) * (NSTREAM * L)
                    v = buf[s2, pl.ds(col, L)]
                    upd = v > am[u]
                    n2.append(jnp.maximum(a2[u], jnp.minimum(v, am[u])))
                    nm.append(jnp.where(upd, v, am[u]))
                    ni.append(jnp.where(
                        upd, gb + (jb + u) * (NSTREAM * L), ai[u]))
                return tuple(nm), tuple(ni), tuple(n2)

            am = tuple(jnp.full((L,), -1.0, jnp.float32)
                       for _ in range(RESCAN_UNROLL))
            ai = tuple(jnp.full((L,), BIG, jnp.int32)
                       for _ in range(RESCAN_UNROLL))
            a2 = tuple(jnp.full((L,), -1.0, jnp.float32)
                       for _ in range(RESCAN_UNROLL))
            am, ai, a2 = lax.fori_loop(0, BAND * VPRS // RESCAN_UNROLL,
                                       rstep, (am, ai, a2))
            am, ai, a2 = list(am), list(ai), list(a2)
            w = RESCAN_UNROLL
            while w > 1:
                w //= 2
                for q in range(w):
                    a2[q] = jnp.maximum(jnp.maximum(a2[q], a2[q + w]),
                                        jnp.minimum(am[q], am[q + w]))
                    am[q], ai[q] = pair_merge(am[q], ai[q],
                                              am[q + w], ai[q + w])
            ms, ixs, m2s = list(ms), list(ixs), list(m2s)
            for p in range(NSTREAM):
                cond = pstar == p
                ms[p] = jnp.where(cond, am[0], ms[p])
                ixs[p] = jnp.where(cond, ai[0], ixs[p])
                m2s[p] = jnp.where(cond, a2[0], m2s[p])
            return tuple(ms), tuple(ixs), tuple(m2s)

        def round_body(r, carry):
            ms, ixs, m2s, cand_v, cand_i = carry
            mC, iC = merge_all(ms, ixs)
            gmax = jnp.max(mC)
            smax = jnp.max(jnp.where(mC == gmax, iC, 0))

            def w_cond(st):
                return st[3] >= S_BASE

            def w_body(st):
                ms, ixs, m2s, _ = st
                pstar = st[3] - S_BASE
                ms, ixs, m2s = refresh(pstar, ms, ixs, m2s)
                mC, iC = merge_all(ms, ixs)
                return (ms, ixs, m2s,
                        jnp.max(jnp.where(mC == gmax, iC, 0)))

            ms, ixs, m2s, _ = lax.while_loop(w_cond, w_body,
                                             (ms, ixs, m2s, smax))
            mC, iC = merge_all(ms, ixs)
            cidx = jnp.min(jnp.where(mC == gmax, iC, BIG))
            cand_v = jnp.where(iota == r, gmax, cand_v)
            cand_i = jnp.where(iota == r, cidx, cand_i)

            srow = lax.shift_right_logical(cidx, 15) - row0
            scol = (cidx & (COLS - 1)) - ccol0
            plsc.store_scatter(buf,
                               [jnp.full((L,), srow, jnp.int32),
                                jnp.full((L,), scol, jnp.int32)],
                               jnp.full((L,), -1.0, jnp.float32),
                               mask=iota == 0)
            pw = lax.shift_right_logical(scol, 4) & (NSTREAM - 1)
            lw = iota == (scol & (L - 1))
            ms, ixs, m2s = list(ms), list(ixs), list(m2s)
            for p in range(NSTREAM):
                cpv = lw & (pw == p)
                ms[p] = jnp.where(cpv, m2s[p], ms[p])
                ixs[p] = jnp.where(cpv, S_BASE + p, ixs[p])
                m2s[p] = jnp.where(cpv, -1.0, m2s[p])
            return tuple(ms), tuple(ixs), tuple(m2s), cand_v, cand_i

        cand_v = jnp.full((L,), -1.0, jnp.float32)
        cand_i = jnp.zeros((L,), jnp.int32)
        ms, ixs, m2s, cand_v, cand_i = lax.fori_loop(
            0, K, round_body, (ms, ixs, m2s, cand_v, cand_i))

        stage[pl.ds(c * L, L)] = cand_v
        stage[pl.ds(CWORDS + c * L, L)] = plsc.bitcast(cand_i, jnp.float32)

    pltpu.sync_copy(stage, pk_ref.at[wid])


_sc_topk = functools.partial(
    pl.kernel,
    out_type=jax.ShapeDtypeStruct((NW, 2 * CWORDS), jnp.float32),
    mesh=plsc.VectorSubcoreMesh(core_axis_name="c", subcore_axis_name="s"),
    compiler_params=pltpu.CompilerParams(needs_layout_passes=False,
                                         use_tc_tiling_on_sc=True),
    scratch_types=[
        pltpu.VMEM((BAND, CCOLS), jnp.float32),
        pltpu.VMEM((BAND, CCOLS), jnp.float32),
        pltpu.VMEM((2 * CWORDS,), jnp.float32),
        pltpu.SemaphoreType.DMA,
        pltpu.SemaphoreType.DMA,
    ],
)(_sc_topk_body)

SUMCOLS = COLS // 8


def _tc_sum_body(x_ref, o_ref):
    i = pl.program_id(0)
    x = x_ref[...]
    acc = jnp.zeros((ROWS, 128), jnp.float32)
    for j in range(SUMCOLS // 128):
        acc = acc + x[:, j * 128:(j + 1) * 128]

    @pl.when(i == 0)
    def _init():
        o_ref[...] = acc

    @pl.when(i > 0)
    def _accum():
        o_ref[...] = o_ref[...] + acc


def _merge_body(pk_ref, s_ref, o_ref):
    pk = pk_ref[...]
    v = pk[:, :CWORDS]
    ix = lax.bitcast_convert_type(pk[:, CWORDS:], jnp.int32)
    total = jnp.sum(s_ref[...])
    kt = jnp.minimum(jnp.float32(K), total).astype(jnp.int32)
    for r in range(K):
        gmax = jnp.max(v)
        gi = jnp.min(jnp.where(v == gmax, ix, BIG))
        valid = r < kt
        row = lax.shift_right_logical(gi, 15)
        col = gi & (COLS - 1)
        o_ref[r, 0] = jnp.where(valid, row, 0)
        o_ref[r, 1] = jnp.where(valid, col, 0)
        v = jnp.where((ix == gi) & (v == gmax), jnp.float32(-1.0), v)


def kernel(inp):
    x = inp.astype(jnp.float32)
    pk = _sc_topk(x)
    sums = pl.pallas_call(
        _tc_sum_body,
        grid=(COLS // SUMCOLS,),
        in_specs=[pl.BlockSpec((ROWS, SUMCOLS), lambda i: (0, i))],
        out_specs=pl.BlockSpec((ROWS, 128), lambda i: (0, 0)),
        out_shape=jax.ShapeDtypeStruct((ROWS, 128), jnp.float32),
    )(x)
    out = pl.pallas_call(
        _merge_body,
        out_shape=jax.ShapeDtypeStruct((K, 2), jnp.int32),
        out_specs=pl.BlockSpec(memory_space=pltpu.SMEM),
    )(pk, sums)
    return out.astype(jnp.int64)


# speedup vs baseline: 1.0032x; 1.0032x over previous
import functools

import jax
import jax.numpy as jnp
from jax import lax
from jax.experimental import pallas as pl
from jax.experimental.pallas import tpu as pltpu
from jax.experimental.pallas import tpu_sc as plsc

ROWS = 128
COLS = 32768
K = 10
BIG = 0x7FFFFFFF
S_BASE = 1 << 30

_info = plsc.get_sparse_core_info()
NC = _info.num_cores
NS = _info.num_subcores
L = _info.num_lanes
NW = NC * NS
BAND = 8
NBAND = ROWS // BAND
COLS_W = COLS // (NW // NBAND)
NCHUNK = 4
CCOLS = COLS_W // NCHUNK
VECS = CCOLS // L
UNROLL = 16
NSTREAM = 8
VPRS = VECS // NSTREAM
RESCAN_UNROLL = 4
CWORDS = NCHUNK * L


def _sc_topk_body(inp_ref, pk_ref, buf0, buf1, stage, sem0, sem1):
    iota = lax.iota(jnp.int32, L)
    wid = lax.axis_index("s") * NC + lax.axis_index("c")
    band = wid // 2
    row0 = band * BAND
    col0 = (wid & 1) * COLS_W
    bufs = (buf0, buf1)
    sems = (sem0, sem1)

    def chunk_src(c):
        return inp_ref.at[pl.ds(row0, BAND), pl.ds(col0 + c * CCOLS, CCOLS)]

    copies = [None] * NCHUNK
    copies[0] = pltpu.make_async_copy(chunk_src(0), bufs[0], sems[0])
    copies[0].start()

    for c in range(NCHUNK):
        if c + 1 < NCHUNK:
            copies[c + 1] = pltpu.make_async_copy(
                chunk_src(c + 1), bufs[(c + 1) & 1], sems[(c + 1) & 1])
            copies[c + 1].start()
        copies[c].wait()
        buf = bufs[c & 1]
        ccol0 = col0 + c * CCOLS

        GPR = VECS // UNROLL

        def scan_step(i, carry, buf=buf, ccol0=ccol0):
            ms, ixs, m2s = carry
            ms, ixs, m2s = list(ms), list(ixs), list(m2s)
            s = i // GPR
            colb = (i % GPR) * (L * UNROLL)
            gvec = (row0 + s) * COLS + ccol0 + colb + iota
            for u in range(UNROLL):
                p = u % NSTREAM
                v = buf[s, pl.ds(colb + u * L, L)]
                upd = v > ms[p]
                m2s[p] = jnp.maximum(m2s[p], jnp.minimum(v, ms[p]))
                ms[p] = jnp.where(upd, v, ms[p])
                ixs[p] = jnp.where(upd, gvec, ixs[p])
                gvec = gvec + L
            return tuple(ms), tuple(ixs), tuple(m2s)

        ms = tuple(jnp.full((L,), -1.0, jnp.float32)
                   for _ in range(NSTREAM))
        ixs = tuple(jnp.full((L,), BIG, jnp.int32) for _ in range(NSTREAM))
        m2s = tuple(jnp.full((L,), -1.0, jnp.float32)
                    for _ in range(NSTREAM))
        ms, ixs, m2s = lax.fori_loop(0, BAND * GPR, scan_step,
                                     (ms, ixs, m2s))

        def pair_merge(m1, i1, m2, i2):
            tk = (m2 > m1) | ((m2 == m1) & (i2 < i1))
            return jnp.where(tk, m2, m1), jnp.where(tk, i2, i1)

        def merge_all(ms, ixs):
            mC, iC = ms[0], ixs[0]
            for p in range(1, NSTREAM):
                mC, iC = pair_merge(mC, iC, ms[p], ixs[p])
            return mC, iC

        def refresh(pstar, ms, ixs, m2s, buf=buf, ccol0=ccol0):
            svid = pstar * L

            def rstep(k, carry):
                am, ai, a2 = carry
                nm, ni, n2 = [], [], []
                s2 = k // (VPRS // RESCAN_UNROLL)
                jb = (k % (VPRS // RESCAN_UNROLL)) * RESCAN_UNROLL
                gb = (row0 + s2) * COLS + ccol0 + svid + iota
                for u in range(RESCAN_UNROLL):
                    col = svid + (jb + u) * (NSTREAM * L)
                    v = buf[s2, pl.ds(col, L)]
                    upd = v > am[u]
                    n2.append(jnp.maximum(a2[u], jnp.minimum(v, am[u])))
                    nm.append(jnp.where(upd, v, am[u]))
                    ni.append(jnp.where(
                        upd, gb + (jb + u) * (NSTREAM * L), ai[u]))
                return tuple(nm), tuple(ni), tuple(n2)

            am = tuple(jnp.full((L,), -1.0, jnp.float32)
                       for _ in range(RESCAN_UNROLL))
            ai = tuple(jnp.full((L,), BIG, jnp.int32)
                       for _ in range(RESCAN_UNROLL))
            a2 = tuple(jnp.full((L,), -1.0, jnp.float32)
                       for _ in range(RESCAN_UNROLL))
            am, ai, a2 = lax.fori_loop(0, BAND * VPRS // RESCAN_UNROLL,
                                       rstep, (am, ai, a2))
            am, ai, a2 = list(am), list(ai), list(a2)
            w = RESCAN_UNROLL
            while w > 1:
                w //= 2
                for q in range(w):
                    a2[q] = jnp.maximum(jnp.maximum(a2[q], a2[q + w]),
                                        jnp.minimum(am[q], am[q + w]))
                    am[q], ai[q] = pair_merge(am[q], ai[q],
                                              am[q + w], ai[q + w])
            ms, ixs, m2s = list(ms), list(ixs), list(m2s)
            for p in range(NSTREAM):
                cond = pstar == p
                ms[p] = jnp.where(cond, am[0], ms[p])
                ixs[p] = jnp.where(cond, ai[0], ixs[p])
                m2s[p] = jnp.where(cond, a2[0], m2s[p])
            return tuple(ms), tuple(ixs), tuple(m2s)

        def round_body(r, carry):
            ms, ixs, m2s, cand_v, cand_i = carry
            mC, iC = merge_all(ms, ixs)
            gmax = jnp.max(mC)
            smax = jnp.max(jnp.where(mC == gmax, iC, 0))

            def w_cond(st):
                return st[3] >= S_BASE

            def w_body(st):
                ms, ixs, m2s, _ = st
                pstar = st[3] - S_BASE
                ms, ixs, m2s = refresh(pstar, ms, ixs, m2s)
                mC, iC = merge_all(ms, ixs)
                return (ms, ixs, m2s,
                        jnp.max(jnp.where(mC == gmax, iC, 0)))

            ms, ixs, m2s, _ = lax.while_loop(w_cond, w_body,
                                             (ms, ixs, m2s, smax))
            mC, iC = merge_all(ms, ixs)
            cidx = jnp.min(jnp.where(mC == gmax, iC, BIG))
            cand_v = jnp.where(iota == r, gmax, cand_v)
            cand_i = jnp.where(iota == r, cidx, cand_i)

            srow = lax.shift_right_logical(cidx, 15) - row0
            scol = (cidx & (COLS - 1)) - ccol0
            plsc.store_scatter(buf,
                               [jnp.full((L,), srow, jnp.int32),
                                jnp.full((L,), scol, jnp.int32)],
                               jnp.full((L,), -1.0, jnp.float32),
                               mask=iota == 0)
            pw = lax.shift_right_logical(scol, 4) & (NSTREAM - 1)
            lw = iota == (scol & (L - 1))
            ms, ixs, m2s = list(ms), list(ixs), list(m2s)
            for p in range(NSTREAM):
                cpv = lw & (pw == p)
                ms[p] = jnp.where(cpv, m2s[p], ms[p])
                ixs[p] = jnp.where(cpv, S_BASE + p, ixs[p])
                m2s[p] = jnp.where(cpv, -1.0, m2s[p])
            return tuple(ms), tuple(ixs), tuple(m2s), cand_v, cand_i

        cand_v = jnp.full((L,), -1.0, jnp.float32)
        cand_i = jnp.zeros((L,), jnp.int32)
        ms, ixs, m2s, cand_v, cand_i = lax.fori_loop(
            0, K, round_body, (ms, ixs, m2s, cand_v, cand_i))

        stage[pl.ds(c * L, L)] = cand_v
        stage[pl.ds(CWORDS + c * L, L)] = plsc.bitcast(cand_i, jnp.float32)

    pltpu.sync_copy(stage, pk_ref.at[wid])


_sc_topk = functools.partial(
    pl.kernel,
    out_type=jax.ShapeDtypeStruct((NW, 2 * CWORDS), jnp.float32),
    mesh=plsc.VectorSubcoreMesh(core_axis_name="c", subcore_axis_name="s"),
    compiler_params=pltpu.CompilerParams(needs_layout_passes=False,
                                         use_tc_tiling_on_sc=True),
    scratch_types=[
        pltpu.VMEM((BAND, CCOLS), jnp.float32),
        pltpu.VMEM((BAND, CCOLS), jnp.float32),
        pltpu.VMEM((2 * CWORDS,), jnp.float32),
        pltpu.SemaphoreType.DMA,
        pltpu.SemaphoreType.DMA,
    ],
)(_sc_topk_body)

SUMCOLS = COLS // 8


def _tc_sum_body(x_ref, o_ref):
    i = pl.program_id(0)
    x = x_ref[...]
    acc = jnp.zeros((ROWS, 128), jnp.float32)
    for j in range(SUMCOLS // 128):
        acc = acc + x[:, j * 128:(j + 1) * 128]

    @pl.when(i == 0)
    def _init():
        o_ref[...] = acc

    @pl.when(i > 0)
    def _accum():
        o_ref[...] = o_ref[...] + acc


def _merge_body(pk_ref, s_ref, o_ref):
    pk = pk_ref[...]
    v = pk[:, :CWORDS]
    ix = lax.bitcast_convert_type(pk[:, CWORDS:], jnp.int32)
    total = jnp.sum(s_ref[...])
    kt = jnp.minimum(jnp.float32(K), total).astype(jnp.int32)
    for r in range(K):
        gmax = jnp.max(v)
        gi = jnp.min(jnp.where(v == gmax, ix, BIG))
        valid = r < kt
        row = lax.shift_right_logical(gi, 15)
        col = gi & (COLS - 1)
        o_ref[r, 0] = jnp.where(valid, row, 0)
        o_ref[r, 1] = jnp.where(valid, col, 0)
        v = jnp.where((ix == gi) & (v == gmax), jnp.float32(-1.0), v)


def kernel(inp):
    x = inp.astype(jnp.float32)
    pk = _sc_topk(x)
    sums = pl.pallas_call(
        _tc_sum_body,
        grid=(COLS // SUMCOLS,),
        in_specs=[pl.BlockSpec((ROWS, SUMCOLS), lambda i: (0, i))],
        out_specs=pl.BlockSpec((ROWS, 128), lambda i: (0, 0)),
        out_shape=jax.ShapeDtypeStruct((ROWS, 128), jnp.float32),
    )(x)
    out = pl.pallas_call(
        _merge_body,
        out_shape=jax.ShapeDtypeStruct((K, 2), jnp.int32),
        out_specs=pl.BlockSpec(memory_space=pltpu.SMEM),
    )(pk, sums)
    return out.astype(jnp.int64)
